# Initial kernel scaffold; baseline (speedup 1.0000x reference)
import jax
import jax.numpy as jnp
from jax.experimental import pallas as pl


def kernel(x, edge_index, W1, b1, W2, b2):
    raise NotImplementedError("write your pallas kernel here")


# speedup vs baseline: 24.6877x; 24.6877x over previous
import functools

import jax
import jax.numpy as jnp
from jax import lax
from jax.experimental import pallas as pl
from jax.experimental.pallas import tpu as pltpu
from jax.experimental.pallas import tpu_sc as plsc

NC = 2
NS = 16
LANES = 128


def _deg_body(dst2, degp, acc, zbuf, ones, didx, sem):
    del sem
    c = lax.axis_index("c")
    s = lax.axis_index("s")
    wid = s * NC + c
    acc_rows = acc.shape[0]
    zt = zbuf.shape[0]

    z16 = jnp.zeros((16,), jnp.float32)

    def zb(i, _):
        zbuf[pl.ds(i * 16, 16)] = z16
        return 0

    lax.fori_loop(0, zt // 16, zb, 0)

    def ob(i, _):
        ones[pl.ds(i * 16, 16)] = z16 + 1.0
        return 0

    lax.fori_loop(0, 128 // 16, ob, 0)

    pltpu.sync_copy(zbuf, acc.at[pl.ds(s * zt, zt)])
    plsc.subcore_barrier()

    rows_per_tile = dst2.shape[0] // (NC * NS)
    row0 = wid * rows_per_tile

    def outer(g, _):
        pltpu.sync_copy(dst2.at[pl.ds(row0 + g * 4, 4)], didx)
        for j in range(4):
            pltpu.sync_copy(ones, acc.at[didx.at[j]], add=True)
        return 0

    lax.fori_loop(0, rows_per_tile // 4, outer, 0)
    plsc.subcore_barrier()

    wt = acc_rows // NS

    @pl.when(c == 0)
    def _():
        pltpu.sync_copy(acc.at[pl.ds(s * wt, wt)], degp.at[0, pl.ds(s * wt, wt)])

    @pl.when(c == 1)
    def _():
        pltpu.sync_copy(acc.at[pl.ds(s * wt, wt)], degp.at[1, pl.ds(s * wt, wt)])


def _gather_body(hs0, hs1, src2, dst2, out_s, acc, rows, sidx, didx, sem):
    c = lax.axis_index("c")
    s = lax.axis_index("s")
    acc_rows = acc.shape[0]

    z16 = jnp.zeros((16,), jnp.float32)

    def zrow(r, _):
        rows[r, pl.ds(0, 16)] = z16
        rows[r, pl.ds(16, 16)] = z16
        return 0

    lax.fori_loop(0, 128, zrow, 0)

    zt = acc_rows // NS

    def zcopy(i, _):
        pltpu.sync_copy(rows, acc.at[pl.ds(s * zt + i * 128, 128)])
        return 0

    lax.fori_loop(0, zt // 128, zcopy, 0)
    plsc.subcore_barrier()

    rows_per_tile = src2.shape[0] // NS
    row0 = s * rows_per_tile

    def run(hs_ref):
        def outer(g, _):
            pltpu.sync_copy(src2.at[pl.ds(row0 + g * 8, 8)], sidx)
            pltpu.sync_copy(dst2.at[pl.ds(row0 + g * 8, 8)], didx)
            for j in range(8):
                pltpu.async_copy(hs_ref.at[sidx.at[j]], rows, sem).wait()
                pltpu.sync_copy(rows, acc.at[didx.at[j]], add=True)
            return 0

        lax.fori_loop(0, rows_per_tile // 8, outer, 0)

    @pl.when(c == 0)
    def _():
        run(hs0)

    @pl.when(c == 1)
    def _():
        run(hs1)

    plsc.subcore_barrier()
    wt = acc_rows // NS

    @pl.when(c == 0)
    def _():
        pltpu.sync_copy(acc.at[pl.ds(s * wt, wt)],
                        out_s.at[0, pl.ds(s * wt, wt)])

    @pl.when(c == 1)
    def _():
        pltpu.sync_copy(acc.at[pl.ds(s * wt, wt)],
                        out_s.at[1, pl.ds(s * wt, wt)])


def _proj_body(x_ref, w1_ref, degp_ref, hs0_ref, hs1_ref):
    h = jnp.dot(x_ref[...], w1_ref[...], preferred_element_type=jnp.float32)
    deg = degp_ref[:, 0] + degp_ref[:, 1] + 1.0
    dis = lax.rsqrt(deg)
    hs = h * dis[:, None]
    hs0_ref[...] = hs[:, :32]
    hs1_ref[...] = hs[:, 32:]


def _epi_body(s_ref, hs0_ref, hs1_ref, degp_ref, b1_ref, w2_ref, b2_ref,
              out_ref):
    deg = degp_ref[:, 0] + degp_ref[:, 1] + 1.0
    dis = lax.rsqrt(deg)
    sfull = jnp.concatenate([s_ref[0] + hs0_ref[...],
                             s_ref[1] + hs1_ref[...]], axis=1)
    agg = sfull * dis[:, None] + b1_ref[...]
    hr = jnp.maximum(agg, 0.0)
    out_ref[...] = (
        jnp.dot(hr, w2_ref[...], preferred_element_type=jnp.float32)
        + b2_ref[...])


def kernel(x, edge_index, W1, b1, W2, b2):
    n = x.shape[0]
    e = edge_index.shape[1]
    d_hid = W1.shape[1]
    d_out = W2.shape[1]
    half = d_hid // 2

    group = LANES * NS * 8
    ep = ((e + group - 1) // group) * group
    acc_rows = ((n + 1 + NS * 128 - 1) // (NS * 128)) * (NS * 128)

    src = edge_index[0]
    dst = edge_index[1]
    pad = ep - e
    src_p = jnp.concatenate([src, jnp.zeros((pad,), jnp.int32)])
    dst_p = jnp.concatenate([dst, jnp.full((pad,), n, jnp.int32)])
    src2 = src_p.reshape(ep // LANES, LANES)
    dst2 = dst_p.reshape(ep // LANES, LANES)

    mesh = plsc.VectorSubcoreMesh(core_axis_name="c", subcore_axis_name="s")
    sc_params = pltpu.CompilerParams(use_tc_tiling_on_sc=False)

    degp = pl.kernel(
        _deg_body,
        out_type=jax.ShapeDtypeStruct((2, acc_rows), jnp.float32),
        mesh=mesh,
        scratch_types=[
            pltpu.VMEM_SHARED((acc_rows,), jnp.float32),
            pltpu.VMEM((acc_rows // NS,), jnp.float32),
            pltpu.VMEM((LANES,), jnp.float32),
            pltpu.VMEM((4, LANES), jnp.int32),
            pltpu.SemaphoreType.DMA,
        ],
        compiler_params=sc_params,
    )(dst2)

    deg_t = jnp.transpose(degp)

    rb = 1000
    n_blocks = n // rb
    hs0, hs1 = pl.pallas_call(
        _proj_body,
        grid=(n_blocks,),
        in_specs=[
            pl.BlockSpec((rb, x.shape[1]), lambda j: (j, 0)),
            pl.BlockSpec((x.shape[1], d_hid), lambda j: (0, 0)),
            pl.BlockSpec((rb, 2), lambda j: (j, 0)),
        ],
        out_specs=[
            pl.BlockSpec((rb, half), lambda j: (j, 0)),
            pl.BlockSpec((rb, half), lambda j: (j, 0)),
        ],
        out_shape=[
            jax.ShapeDtypeStruct((n, half), jnp.float32),
            jax.ShapeDtypeStruct((n, half), jnp.float32),
        ],
    )(x, W1, deg_t)

    s_agg = pl.kernel(
        _gather_body,
        out_type=jax.ShapeDtypeStruct((2, acc_rows, half), jnp.float32),
        mesh=mesh,
        scratch_types=[
            pltpu.VMEM_SHARED((acc_rows, half), jnp.float32),
            pltpu.VMEM((LANES, half), jnp.float32),
            pltpu.VMEM((8, LANES), jnp.int32),
            pltpu.VMEM((8, LANES), jnp.int32),
            pltpu.SemaphoreType.DMA,
        ],
        compiler_params=sc_params,
    )(hs0, hs1, src2, dst2)

    out = pl.pallas_call(
        _epi_body,
        grid=(n_blocks,),
        in_specs=[
            pl.BlockSpec((2, rb, half), lambda j: (0, j, 0)),
            pl.BlockSpec((rb, half), lambda j: (j, 0)),
            pl.BlockSpec((rb, half), lambda j: (j, 0)),
            pl.BlockSpec((rb, 2), lambda j: (j, 0)),
            pl.BlockSpec((1, d_hid), lambda j: (0, 0)),
            pl.BlockSpec((d_hid, d_out), lambda j: (0, 0)),
            pl.BlockSpec((1, d_out), lambda j: (0, 0)),
        ],
        out_specs=pl.BlockSpec((rb, d_out), lambda j: (j, 0)),
        out_shape=jax.ShapeDtypeStruct((n, d_out), jnp.float32),
    )(s_agg, hs0, hs1, deg_t, b1.reshape(1, d_hid), W2,
      b2.reshape(1, d_out))
    return out


# speedup vs baseline: 30.3851x; 1.2308x over previous
import functools

import jax
import jax.numpy as jnp
from jax import lax
from jax.experimental import pallas as pl
from jax.experimental.pallas import tpu as pltpu
from jax.experimental.pallas import tpu_sc as plsc

NC = 2
NS = 16
LANES = 128


def _deg_body(dst2, degp, acc, zbuf, ones, didx, sem):
    c = lax.axis_index("c")
    s = lax.axis_index("s")
    wid = s * NC + c
    acc_rows = acc.shape[0]
    zt = zbuf.shape[0]

    z16 = jnp.zeros((16,), jnp.float32)

    def zb(i, _):
        zbuf[pl.ds(i * 16, 16)] = z16
        return 0

    lax.fori_loop(0, zt // 16, zb, 0)

    def ob(i, _):
        ones[pl.ds(i * 16, 16)] = z16 + 1.0
        return 0

    lax.fori_loop(0, 128 // 16, ob, 0)

    pltpu.sync_copy(zbuf, acc.at[pl.ds(s * zt, zt)])
    plsc.subcore_barrier()

    rows_per_tile = dst2.shape[0] // (NC * NS)
    row0 = wid * rows_per_tile
    ng = rows_per_tile // 4

    pltpu.sync_copy(dst2.at[pl.ds(row0, 4)], didx.at[0])

    def outer(g, _):
        p = lax.rem(g, 2)
        q = 1 - p

        @pl.when(g + 1 < ng)
        def _():
            pltpu.async_copy(dst2.at[pl.ds(row0 + (g + 1) * 4, 4)],
                             didx.at[q], sem)

        for j in range(4):
            pltpu.sync_copy(ones, acc.at[didx.at[p, j]], add=True)

        @pl.when(g + 1 < ng)
        def _():
            pltpu.make_async_copy(dst2.at[pl.ds(0, 4)], didx.at[q],
                                  sem).wait()
        return 0

    lax.fori_loop(0, ng, outer, 0)
    plsc.subcore_barrier()

    wt = acc_rows // NS

    @pl.when(c == 0)
    def _():
        pltpu.sync_copy(acc.at[pl.ds(s * wt, wt)], degp.at[0, pl.ds(s * wt, wt)])

    @pl.when(c == 1)
    def _():
        pltpu.sync_copy(acc.at[pl.ds(s * wt, wt)], degp.at[1, pl.ds(s * wt, wt)])


def _gather_body(hs0, hs1, src2, dst2, out_s, acc, rows, sidx, didx,
                 gsem, isem):
    c = lax.axis_index("c")
    s = lax.axis_index("s")
    acc_rows = acc.shape[0]

    z16 = jnp.zeros((16,), jnp.float32)

    def zrow(r, _):
        rows[0, r, pl.ds(0, 16)] = z16
        rows[0, r, pl.ds(16, 16)] = z16
        return 0

    lax.fori_loop(0, 128, zrow, 0)

    zt = acc_rows // NS

    def zcopy(i, _):
        pltpu.sync_copy(rows.at[0], acc.at[pl.ds(s * zt + i * 128, 128)])
        return 0

    lax.fori_loop(0, zt // 128, zcopy, 0)
    plsc.subcore_barrier()

    rows_per_tile = src2.shape[0] // NS
    row0 = s * rows_per_tile
    ng = rows_per_tile // 8

    def run(hs_ref):
        pltpu.sync_copy(src2.at[pl.ds(row0, 8)], sidx.at[0])
        pltpu.sync_copy(dst2.at[pl.ds(row0, 8)], didx.at[0])
        pltpu.async_copy(hs_ref.at[sidx.at[0, 0]], rows.at[0], gsem)

        def group(g, _):
            p = lax.rem(g, 2)
            q = 1 - p

            @pl.when(g + 1 < ng)
            def _():
                pltpu.async_copy(src2.at[pl.ds(row0 + (g + 1) * 8, 8)],
                                 sidx.at[q], isem)
                pltpu.async_copy(dst2.at[pl.ds(row0 + (g + 1) * 8, 8)],
                                 didx.at[q], isem)

            for j in range(8):
                jb = j & 1
                pltpu.make_async_copy(hs_ref.at[sidx.at[p, j]],
                                      rows.at[jb], gsem).wait()
                if j < 7:
                    pltpu.async_copy(hs_ref.at[sidx.at[p, j + 1]],
                                     rows.at[1 - jb], gsem)
                else:
                    @pl.when(g + 1 < ng)
                    def _():
                        pltpu.make_async_copy(src2.at[pl.ds(0, 8)],
                                              sidx.at[q], isem).wait()
                        pltpu.make_async_copy(dst2.at[pl.ds(0, 8)],
                                              didx.at[q], isem).wait()
                        pltpu.async_copy(hs_ref.at[sidx.at[q, 0]],
                                         rows.at[1 - jb], gsem)
                pltpu.sync_copy(rows.at[jb], acc.at[didx.at[p, j]], add=True)
            return 0

        lax.fori_loop(0, ng, group, 0)

    @pl.when(c == 0)
    def _():
        run(hs0)

    @pl.when(c == 1)
    def _():
        run(hs1)

    plsc.subcore_barrier()
    wt = acc_rows // NS

    @pl.when(c == 0)
    def _():
        pltpu.sync_copy(acc.at[pl.ds(s * wt, wt)],
                        out_s.at[0, pl.ds(s * wt, wt)])

    @pl.when(c == 1)
    def _():
        pltpu.sync_copy(acc.at[pl.ds(s * wt, wt)],
                        out_s.at[1, pl.ds(s * wt, wt)])


def _proj_body(x_ref, w1_ref, degp_ref, hs0_ref, hs1_ref):
    h = jnp.dot(x_ref[...], w1_ref[...], preferred_element_type=jnp.float32)
    deg = degp_ref[:, 0] + degp_ref[:, 1] + 1.0
    dis = lax.rsqrt(deg)
    hs = h * dis[:, None]
    hs0_ref[...] = hs[:, :32]
    hs1_ref[...] = hs[:, 32:]


def _epi_body(s_ref, hs0_ref, hs1_ref, degp_ref, b1_ref, w2_ref, b2_ref,
              out_ref):
    deg = degp_ref[:, 0] + degp_ref[:, 1] + 1.0
    dis = lax.rsqrt(deg)
    sfull = jnp.concatenate([s_ref[0] + hs0_ref[...],
                             s_ref[1] + hs1_ref[...]], axis=1)
    agg = sfull * dis[:, None] + b1_ref[...]
    hr = jnp.maximum(agg, 0.0)
    out_ref[...] = (
        jnp.dot(hr, w2_ref[...], preferred_element_type=jnp.float32)
        + b2_ref[...])


def kernel(x, edge_index, W1, b1, W2, b2):
    n = x.shape[0]
    e = edge_index.shape[1]
    d_hid = W1.shape[1]
    d_out = W2.shape[1]
    half = d_hid // 2

    group = LANES * NS * 8
    ep = ((e + group - 1) // group) * group
    acc_rows = ((n + 1 + NS * 128 - 1) // (NS * 128)) * (NS * 128)

    src = edge_index[0]
    dst = edge_index[1]
    pad = ep - e
    src_p = jnp.concatenate([src, jnp.zeros((pad,), jnp.int32)])
    dst_p = jnp.concatenate([dst, jnp.full((pad,), n, jnp.int32)])
    src2 = src_p.reshape(ep // LANES, LANES)
    dst2 = dst_p.reshape(ep // LANES, LANES)

    mesh = plsc.VectorSubcoreMesh(core_axis_name="c", subcore_axis_name="s")
    sc_params = pltpu.CompilerParams(use_tc_tiling_on_sc=False)

    degp = pl.kernel(
        _deg_body,
        out_type=jax.ShapeDtypeStruct((2, acc_rows), jnp.float32),
        mesh=mesh,
        scratch_types=[
            pltpu.VMEM_SHARED((acc_rows,), jnp.float32),
            pltpu.VMEM((acc_rows // NS,), jnp.float32),
            pltpu.VMEM((LANES,), jnp.float32),
            pltpu.VMEM((2, 4, LANES), jnp.int32),
            pltpu.SemaphoreType.DMA,
        ],
        compiler_params=sc_params,
    )(dst2)

    deg_t = jnp.transpose(degp)

    rb = 1000
    n_blocks = n // rb
    hs0, hs1 = pl.pallas_call(
        _proj_body,
        grid=(n_blocks,),
        in_specs=[
            pl.BlockSpec((rb, x.shape[1]), lambda j: (j, 0)),
            pl.BlockSpec((x.shape[1], d_hid), lambda j: (0, 0)),
            pl.BlockSpec((rb, 2), lambda j: (j, 0)),
        ],
        out_specs=[
            pl.BlockSpec((rb, half), lambda j: (j, 0)),
            pl.BlockSpec((rb, half), lambda j: (j, 0)),
        ],
        out_shape=[
            jax.ShapeDtypeStruct((n, half), jnp.float32),
            jax.ShapeDtypeStruct((n, half), jnp.float32),
        ],
    )(x, W1, deg_t)

    s_agg = pl.kernel(
        _gather_body,
        out_type=jax.ShapeDtypeStruct((2, acc_rows, half), jnp.float32),
        mesh=mesh,
        scratch_types=[
            pltpu.VMEM_SHARED((acc_rows, half), jnp.float32),
            pltpu.VMEM((2, LANES, half), jnp.float32),
            pltpu.VMEM((2, 8, LANES), jnp.int32),
            pltpu.VMEM((2, 8, LANES), jnp.int32),
            pltpu.SemaphoreType.DMA,
            pltpu.SemaphoreType.DMA,
        ],
        compiler_params=sc_params,
    )(hs0, hs1, src2, dst2)

    out = pl.pallas_call(
        _epi_body,
        grid=(n_blocks,),
        in_specs=[
            pl.BlockSpec((2, rb, half), lambda j: (0, j, 0)),
            pl.BlockSpec((rb, half), lambda j: (j, 0)),
            pl.BlockSpec((rb, half), lambda j: (j, 0)),
            pl.BlockSpec((rb, 2), lambda j: (j, 0)),
            pl.BlockSpec((1, d_hid), lambda j: (0, 0)),
            pl.BlockSpec((d_hid, d_out), lambda j: (0, 0)),
            pl.BlockSpec((1, d_out), lambda j: (0, 0)),
        ],
        out_specs=pl.BlockSpec((rb, d_out), lambda j: (j, 0)),
        out_shape=jax.ShapeDtypeStruct((n, d_out), jnp.float32),
    )(s_agg, hs0, hs1, deg_t, b1.reshape(1, d_hid), W2,
      b2.reshape(1, d_out))
    return out


# speedup vs baseline: 33.9416x; 1.1170x over previous
import functools

import jax
import jax.numpy as jnp
from jax import lax
from jax.experimental import pallas as pl
from jax.experimental.pallas import tpu as pltpu
from jax.experimental.pallas import tpu_sc as plsc

NC = 2
NS = 16
LANES = 128


def _deg_body(dst2, degp, acc, zbuf, ones, didx, sem):
    c = lax.axis_index("c")
    s = lax.axis_index("s")
    wid = s * NC + c
    acc_rows = acc.shape[0]
    zt = zbuf.shape[0]

    z16 = jnp.zeros((16,), jnp.float32)

    def zb(i, _):
        zbuf[pl.ds(i * 16, 16)] = z16
        return 0

    lax.fori_loop(0, zt // 16, zb, 0)

    def ob(i, _):
        ones[pl.ds(i * 16, 16)] = z16 + 1.0
        return 0

    lax.fori_loop(0, 128 // 16, ob, 0)

    pltpu.sync_copy(zbuf, acc.at[pl.ds(s * zt, zt)])
    plsc.subcore_barrier()

    rows_per_tile = dst2.shape[0] // (NC * NS)
    row0 = wid * rows_per_tile
    ng = rows_per_tile // 4

    pltpu.sync_copy(dst2.at[pl.ds(row0, 4)], didx.at[0])

    def outer(g, _):
        p = lax.rem(g, 2)
        q = 1 - p

        @pl.when(g + 1 < ng)
        def _():
            pltpu.async_copy(dst2.at[pl.ds(row0 + (g + 1) * 4, 4)],
                             didx.at[q], sem)

        for j in range(4):
            pltpu.sync_copy(ones, acc.at[didx.at[p, j]], add=True)

        @pl.when(g + 1 < ng)
        def _():
            pltpu.make_async_copy(dst2.at[pl.ds(0, 4)], didx.at[q],
                                  sem).wait()
        return 0

    lax.fori_loop(0, ng, outer, 0)
    plsc.subcore_barrier()

    wt = acc_rows // NS

    @pl.when(c == 0)
    def _():
        pltpu.sync_copy(acc.at[pl.ds(s * wt, wt)], degp.at[0, pl.ds(s * wt, wt)])

    @pl.when(c == 1)
    def _():
        pltpu.sync_copy(acc.at[pl.ds(s * wt, wt)], degp.at[1, pl.ds(s * wt, wt)])


def _gather_body(hs0, hs1, src2, dst2, out_s, acc, rows, sidx, didx,
                 gsem, isem):
    c = lax.axis_index("c")
    s = lax.axis_index("s")
    acc_rows = acc.shape[0]

    z16 = jnp.zeros((16,), jnp.float32)

    def zrow(r, _):
        rows[0, r, pl.ds(0, 16)] = z16
        rows[0, r, pl.ds(16, 16)] = z16
        return 0

    lax.fori_loop(0, 128, zrow, 0)

    zt = acc_rows // NS

    def zcopy(i, _):
        pltpu.sync_copy(rows.at[0], acc.at[pl.ds(s * zt + i * 128, 128)])
        return 0

    lax.fori_loop(0, zt // 128, zcopy, 0)
    plsc.subcore_barrier()

    rows_per_tile = src2.shape[0] // NS
    row0 = s * rows_per_tile
    ng = rows_per_tile // 8

    def run(hs_ref):
        pltpu.sync_copy(src2.at[pl.ds(row0, 8)], sidx.at[0])
        pltpu.sync_copy(dst2.at[pl.ds(row0, 8)], didx.at[0])
        pltpu.async_copy(hs_ref.at[sidx.at[0, 0]], rows.at[0], gsem)

        def group(g, _):
            p = lax.rem(g, 2)
            q = 1 - p

            @pl.when(g + 1 < ng)
            def _():
                pltpu.async_copy(src2.at[pl.ds(row0 + (g + 1) * 8, 8)],
                                 sidx.at[q], isem)
                pltpu.async_copy(dst2.at[pl.ds(row0 + (g + 1) * 8, 8)],
                                 didx.at[q], isem)

            for j in range(8):
                jb = j & 1
                pltpu.make_async_copy(hs_ref.at[sidx.at[p, j]],
                                      rows.at[jb], gsem).wait()
                if j < 7:
                    pltpu.async_copy(hs_ref.at[sidx.at[p, j + 1]],
                                     rows.at[1 - jb], gsem)
                else:
                    @pl.when(g + 1 < ng)
                    def _():
                        pltpu.make_async_copy(src2.at[pl.ds(0, 8)],
                                              sidx.at[q], isem).wait()
                        pltpu.make_async_copy(dst2.at[pl.ds(0, 8)],
                                              didx.at[q], isem).wait()
                        pltpu.async_copy(hs_ref.at[sidx.at[q, 0]],
                                         rows.at[1 - jb], gsem)
                pltpu.sync_copy(rows.at[jb], acc.at[didx.at[p, j]], add=True)
            return 0

        lax.fori_loop(0, ng, group, 0)

    @pl.when(c == 0)
    def _():
        run(hs0)

    @pl.when(c == 1)
    def _():
        run(hs1)

    plsc.subcore_barrier()
    wt = acc_rows // NS

    @pl.when(c == 0)
    def _():
        pltpu.sync_copy(acc.at[pl.ds(s * wt, wt)],
                        out_s.at[0, pl.ds(s * wt, wt)])

    @pl.when(c == 1)
    def _():
        pltpu.sync_copy(acc.at[pl.ds(s * wt, wt)],
                        out_s.at[1, pl.ds(s * wt, wt)])


def _proj_body(xp_ref, w0_ref, w1_ref, dp_ref, hs0_ref, hs1_ref):
    xb = xp_ref[...]
    d = dp_ref[...]
    hs0_ref[...] = d * jnp.dot(xb, w0_ref[...],
                               preferred_element_type=jnp.float32)
    hs1_ref[...] = d * jnp.dot(xb, w1_ref[...],
                               preferred_element_type=jnp.float32)


def _epi_body(s0_ref, s1_ref, h0_ref, h1_ref, dp_ref, b10_ref, b11_ref,
              w20_ref, w21_ref, b2p_ref, out_ref):
    d = dp_ref[...]
    t0 = jnp.maximum((s0_ref[...] + h0_ref[...]) * d + b10_ref[...], 0.0)
    t1 = jnp.maximum((s1_ref[...] + h1_ref[...]) * d + b11_ref[...], 0.0)
    out_ref[...] = (
        jnp.dot(t0, w20_ref[...], preferred_element_type=jnp.float32)
        + jnp.dot(t1, w21_ref[...], preferred_element_type=jnp.float32)
        + b2p_ref[...])


def kernel(x, edge_index, W1, b1, W2, b2):
    n = x.shape[0]
    e = edge_index.shape[1]
    d_hid = W1.shape[1]
    d_out = W2.shape[1]
    half = d_hid // 2

    group = LANES * NS * 8
    ep = ((e + group - 1) // group) * group
    acc_rows = ((n + 1 + NS * 128 - 1) // (NS * 128)) * (NS * 128)

    src = edge_index[0]
    dst = edge_index[1]
    pad = ep - e
    src_p = jnp.concatenate([src, jnp.zeros((pad,), jnp.int32)])
    dst_p = jnp.concatenate([dst, jnp.full((pad,), n, jnp.int32)])
    src2 = src_p.reshape(ep // LANES, LANES)
    dst2 = dst_p.reshape(ep // LANES, LANES)

    mesh = plsc.VectorSubcoreMesh(core_axis_name="c", subcore_axis_name="s")
    sc_params = pltpu.CompilerParams(use_tc_tiling_on_sc=False)

    degp = pl.kernel(
        _deg_body,
        out_type=jax.ShapeDtypeStruct((2, acc_rows), jnp.float32),
        mesh=mesh,
        scratch_types=[
            pltpu.VMEM_SHARED((acc_rows,), jnp.float32),
            pltpu.VMEM((acc_rows // NS,), jnp.float32),
            pltpu.VMEM((LANES,), jnp.float32),
            pltpu.VMEM((2, 4, LANES), jnp.int32),
            pltpu.SemaphoreType.DMA,
        ],
        compiler_params=sc_params,
    )(dst2)

    pb = 1024
    g = (n + pb - 1) // pb
    np_pad = g * pb
    prows = np_pad // 4

    deg = degp[0, :np_pad] + degp[1, :np_pad] + 1.0
    disp = jnp.repeat(lax.rsqrt(deg), half).reshape(prows, 128)

    x_p = x.reshape(n // 4, 4 * x.shape[1])
    eye4 = jnp.eye(4, dtype=jnp.float32)
    w1b0 = jnp.kron(eye4, W1[:, :half])
    w1b1 = jnp.kron(eye4, W1[:, half:])

    hsp0, hsp1 = pl.pallas_call(
        _proj_body,
        grid=(g,),
        in_specs=[
            pl.BlockSpec((pb // 4, 4 * x.shape[1]), lambda j: (j, 0)),
            pl.BlockSpec((4 * x.shape[1], 128), lambda j: (0, 0)),
            pl.BlockSpec((4 * x.shape[1], 128), lambda j: (0, 0)),
            pl.BlockSpec((pb // 4, 128), lambda j: (j, 0)),
        ],
        out_specs=[
            pl.BlockSpec((pb // 4, 128), lambda j: (j, 0)),
            pl.BlockSpec((pb // 4, 128), lambda j: (j, 0)),
        ],
        out_shape=[
            jax.ShapeDtypeStruct((prows, 128), jnp.float32),
            jax.ShapeDtypeStruct((prows, 128), jnp.float32),
        ],
    )(x_p, w1b0, w1b1, disp)

    hs0_lin = hsp0.reshape(np_pad, half)
    hs1_lin = hsp1.reshape(np_pad, half)

    s_agg = pl.kernel(
        _gather_body,
        out_type=jax.ShapeDtypeStruct((2, acc_rows, half), jnp.float32),
        mesh=mesh,
        scratch_types=[
            pltpu.VMEM_SHARED((acc_rows, half), jnp.float32),
            pltpu.VMEM((2, LANES, half), jnp.float32),
            pltpu.VMEM((2, 8, LANES), jnp.int32),
            pltpu.VMEM((2, 8, LANES), jnp.int32),
            pltpu.SemaphoreType.DMA,
            pltpu.SemaphoreType.DMA,
        ],
        compiler_params=sc_params,
    )(hs0_lin, hs1_lin, src2, dst2)

    s_lin = s_agg.reshape(2 * acc_rows * half // 128, 128)
    off1 = acc_rows * half // 128 // (pb // 4)

    w2b0 = jnp.kron(eye4, W2[:half, :])
    w2b1 = jnp.kron(eye4, W2[half:, :])
    b1p0 = jnp.tile(b1[:half], 4).reshape(1, 128)
    b1p1 = jnp.tile(b1[half:], 4).reshape(1, 128)
    b2p = jnp.tile(b2, 4).reshape(1, 4 * d_out)

    out_p = pl.pallas_call(
        _epi_body,
        grid=(g,),
        in_specs=[
            pl.BlockSpec((pb // 4, 128), lambda j: (j, 0)),
            pl.BlockSpec((pb // 4, 128), lambda j: (j + off1, 0)),
            pl.BlockSpec((pb // 4, 128), lambda j: (j, 0)),
            pl.BlockSpec((pb // 4, 128), lambda j: (j, 0)),
            pl.BlockSpec((pb // 4, 128), lambda j: (j, 0)),
            pl.BlockSpec((1, 128), lambda j: (0, 0)),
            pl.BlockSpec((1, 128), lambda j: (0, 0)),
            pl.BlockSpec((128, 4 * d_out), lambda j: (0, 0)),
            pl.BlockSpec((128, 4 * d_out), lambda j: (0, 0)),
            pl.BlockSpec((1, 4 * d_out), lambda j: (0, 0)),
        ],
        out_specs=pl.BlockSpec((pb // 4, 4 * d_out), lambda j: (j, 0)),
        out_shape=jax.ShapeDtypeStruct((prows, 4 * d_out), jnp.float32),
    )(s_lin, s_lin, hsp0, hsp1, disp, b1p0, b1p1, w2b0, w2b1, b2p)

    return out_p.reshape(np_pad, d_out)[:n]


# speedup vs baseline: 44.8412x; 1.3211x over previous
import functools

import jax
import jax.numpy as jnp
from jax import lax
from jax.experimental import pallas as pl
from jax.experimental.pallas import tpu as pltpu
from jax.experimental.pallas import tpu_sc as plsc

NC = 2
NS = 16
LANES = 128


def _deg_body(dst2, degp, acc, zbuf, ones, didx, sem):
    c = lax.axis_index("c")
    s = lax.axis_index("s")
    wid = s * NC + c
    acc_rows = acc.shape[0]
    zt = zbuf.shape[0]

    z16 = jnp.zeros((16,), jnp.float32)

    def zb(i, _):
        zbuf[pl.ds(i * 16, 16)] = z16
        return 0

    lax.fori_loop(0, zt // 16, zb, 0)

    def ob(i, _):
        ones[pl.ds(i * 16, 16)] = z16 + 1.0
        return 0

    lax.fori_loop(0, 128 // 16, ob, 0)

    pltpu.sync_copy(zbuf, acc.at[pl.ds(s * zt, zt)])
    plsc.subcore_barrier()

    rows_per_tile = dst2.shape[0] // (NC * NS)
    row0 = wid * rows_per_tile
    ng = rows_per_tile // 4

    pltpu.sync_copy(dst2.at[pl.ds(row0, 4)], didx.at[0])

    def outer(g, _):
        p = lax.rem(g, 2)
        q = 1 - p

        @pl.when(g + 1 < ng)
        def _():
            pltpu.async_copy(dst2.at[pl.ds(row0 + (g + 1) * 4, 4)],
                             didx.at[q], sem)

        for j in range(4):
            pltpu.sync_copy(ones, acc.at[didx.at[p, j]], add=True)

        @pl.when(g + 1 < ng)
        def _():
            pltpu.make_async_copy(dst2.at[pl.ds(0, 4)], didx.at[q],
                                  sem).wait()
        return 0

    lax.fori_loop(0, ng, outer, 0)
    plsc.subcore_barrier()

    wt = acc_rows // NS

    @pl.when(c == 0)
    def _():
        pltpu.sync_copy(acc.at[pl.ds(s * wt, wt)], degp.at[0, pl.ds(s * wt, wt)])

    @pl.when(c == 1)
    def _():
        pltpu.sync_copy(acc.at[pl.ds(s * wt, wt)], degp.at[1, pl.ds(s * wt, wt)])


def _gather_body(hs0, hs1, src2, dst2, out_s, acc, rows, sidx, didx,
                 gsem0, gsem1, isem):
    c = lax.axis_index("c")
    s = lax.axis_index("s")
    acc_rows = acc.shape[0]

    z16 = jnp.zeros((16,), jnp.float32)

    def zrow(r, _):
        rows[0, r, pl.ds(0, 16)] = z16
        rows[0, r, pl.ds(16, 16)] = z16
        return 0

    lax.fori_loop(0, 128, zrow, 0)

    zt = acc_rows // NS

    def zcopy(i, _):
        pltpu.sync_copy(rows.at[0], acc.at[pl.ds(s * zt + i * 128, 128)])
        return 0

    lax.fori_loop(0, zt // 128, zcopy, 0)
    plsc.subcore_barrier()

    rows_per_tile = src2.shape[0] // NS
    row0 = s * rows_per_tile
    ng = rows_per_tile // 8

    def run(hs_ref):
        pltpu.sync_copy(src2.at[pl.ds(row0, 8)], sidx.at[0])
        pltpu.sync_copy(dst2.at[pl.ds(row0, 8)], didx.at[0])
        pltpu.async_copy(hs_ref.at[sidx.at[0, 0]], rows.at[0], gsem0)
        pltpu.async_copy(hs_ref.at[sidx.at[0, 1]], rows.at[1], gsem1)

        def group(g, _):
            p = lax.rem(g, 2)
            q = 1 - p

            @pl.when(g + 1 < ng)
            def _():
                pltpu.async_copy(src2.at[pl.ds(row0 + (g + 1) * 8, 8)],
                                 sidx.at[q], isem)
                pltpu.async_copy(dst2.at[pl.ds(row0 + (g + 1) * 8, 8)],
                                 didx.at[q], isem)

            for j in range(8):
                jb = j & 3
                sem = gsem0 if (j & 1) == 0 else gsem1
                pltpu.make_async_copy(hs_ref.at[sidx.at[p, j]],
                                      rows.at[jb], sem).wait()
                if j < 6:
                    pltpu.async_copy(hs_ref.at[sidx.at[p, j + 2]],
                                     rows.at[(j + 2) & 3], sem)
                elif j == 6:
                    @pl.when(g + 1 < ng)
                    def _():
                        pltpu.make_async_copy(src2.at[pl.ds(0, 8)],
                                              sidx.at[q], isem).wait()
                        pltpu.make_async_copy(dst2.at[pl.ds(0, 8)],
                                              didx.at[q], isem).wait()
                        pltpu.async_copy(hs_ref.at[sidx.at[q, 0]],
                                         rows.at[(j + 2) & 3], sem)
                else:
                    @pl.when(g + 1 < ng)
                    def _():
                        pltpu.async_copy(hs_ref.at[sidx.at[q, 1]],
                                         rows.at[(j + 2) & 3], sem)
                pltpu.sync_copy(rows.at[jb], acc.at[didx.at[p, j]], add=True)
            return 0

        lax.fori_loop(0, ng, group, 0)

    @pl.when(c == 0)
    def _():
        run(hs0)

    @pl.when(c == 1)
    def _():
        run(hs1)

    plsc.subcore_barrier()
    wt = acc_rows // NS

    @pl.when(c == 0)
    def _():
        pltpu.sync_copy(acc.at[pl.ds(s * wt, wt)],
                        out_s.at[0, pl.ds(s * wt, wt)])

    @pl.when(c == 1)
    def _():
        pltpu.sync_copy(acc.at[pl.ds(s * wt, wt)],
                        out_s.at[1, pl.ds(s * wt, wt)])


def _proj_body(xp_ref, w0_ref, w1_ref, dp_ref, hs0_ref, hs1_ref):
    xb = xp_ref[...]
    d = dp_ref[...]
    hs0_ref[...] = d * jnp.dot(xb, w0_ref[...],
                               preferred_element_type=jnp.float32)
    hs1_ref[...] = d * jnp.dot(xb, w1_ref[...],
                               preferred_element_type=jnp.float32)


def _epi_body(s0_ref, s1_ref, h0_ref, h1_ref, dp_ref, b10_ref, b11_ref,
              w20_ref, w21_ref, b2p_ref, out_ref):
    d = dp_ref[...]
    t0 = jnp.maximum((s0_ref[...] + h0_ref[...]) * d + b10_ref[...], 0.0)
    t1 = jnp.maximum((s1_ref[...] + h1_ref[...]) * d + b11_ref[...], 0.0)
    out_ref[...] = (
        jnp.dot(t0, w20_ref[...], preferred_element_type=jnp.float32)
        + jnp.dot(t1, w21_ref[...], preferred_element_type=jnp.float32)
        + b2p_ref[...])


def kernel(x, edge_index, W1, b1, W2, b2):
    n = x.shape[0]
    e = edge_index.shape[1]
    d_hid = W1.shape[1]
    d_out = W2.shape[1]
    half = d_hid // 2

    group = LANES * NS * 8
    ep = ((e + group - 1) // group) * group
    acc_rows = ((n + 1 + NS * 128 - 1) // (NS * 128)) * (NS * 128)

    src = edge_index[0]
    dst = edge_index[1]
    pad = ep - e
    src_p = jnp.concatenate([src, jnp.zeros((pad,), jnp.int32)])
    dst_p = jnp.concatenate([dst, jnp.full((pad,), n, jnp.int32)])
    src2 = src_p.reshape(ep // LANES, LANES)
    dst2 = dst_p.reshape(ep // LANES, LANES)

    mesh = plsc.VectorSubcoreMesh(core_axis_name="c", subcore_axis_name="s")
    sc_params = pltpu.CompilerParams(use_tc_tiling_on_sc=False)

    degp = pl.kernel(
        _deg_body,
        out_type=jax.ShapeDtypeStruct((2, acc_rows), jnp.float32),
        mesh=mesh,
        scratch_types=[
            pltpu.VMEM_SHARED((acc_rows,), jnp.float32),
            pltpu.VMEM((acc_rows // NS,), jnp.float32),
            pltpu.VMEM((LANES,), jnp.float32),
            pltpu.VMEM((2, 4, LANES), jnp.int32),
            pltpu.SemaphoreType.DMA,
        ],
        compiler_params=sc_params,
    )(dst2)

    pb = 1024
    g = (n + pb - 1) // pb
    np_pad = g * pb
    prows = np_pad // 4

    deg = degp[0, :np_pad] + degp[1, :np_pad] + 1.0
    disp = jnp.repeat(lax.rsqrt(deg), half).reshape(prows, 128)

    x_p = x.reshape(n // 4, 4 * x.shape[1])
    eye4 = jnp.eye(4, dtype=jnp.float32)
    w1b0 = jnp.kron(eye4, W1[:, :half])
    w1b1 = jnp.kron(eye4, W1[:, half:])

    hsp0, hsp1 = pl.pallas_call(
        _proj_body,
        grid=(g,),
        in_specs=[
            pl.BlockSpec((pb // 4, 4 * x.shape[1]), lambda j: (j, 0)),
            pl.BlockSpec((4 * x.shape[1], 128), lambda j: (0, 0)),
            pl.BlockSpec((4 * x.shape[1], 128), lambda j: (0, 0)),
            pl.BlockSpec((pb // 4, 128), lambda j: (j, 0)),
        ],
        out_specs=[
            pl.BlockSpec((pb // 4, 128), lambda j: (j, 0)),
            pl.BlockSpec((pb // 4, 128), lambda j: (j, 0)),
        ],
        out_shape=[
            jax.ShapeDtypeStruct((prows, 128), jnp.float32),
            jax.ShapeDtypeStruct((prows, 128), jnp.float32),
        ],
    )(x_p, w1b0, w1b1, disp)

    hs0_lin = hsp0.reshape(np_pad, half)
    hs1_lin = hsp1.reshape(np_pad, half)

    s_agg = pl.kernel(
        _gather_body,
        out_type=jax.ShapeDtypeStruct((2, acc_rows, half), jnp.float32),
        mesh=mesh,
        scratch_types=[
            pltpu.VMEM_SHARED((acc_rows, half), jnp.float32),
            pltpu.VMEM((4, LANES, half), jnp.float32),
            pltpu.VMEM((2, 8, LANES), jnp.int32),
            pltpu.VMEM((2, 8, LANES), jnp.int32),
            pltpu.SemaphoreType.DMA,
            pltpu.SemaphoreType.DMA,
            pltpu.SemaphoreType.DMA,
        ],
        compiler_params=sc_params,
    )(hs0_lin, hs1_lin, src2, dst2)

    s_lin = s_agg.reshape(2 * acc_rows * half // 128, 128)
    off1 = acc_rows * half // 128 // (pb // 4)

    w2b0 = jnp.kron(eye4, W2[:half, :])
    w2b1 = jnp.kron(eye4, W2[half:, :])
    b1p0 = jnp.tile(b1[:half], 4).reshape(1, 128)
    b1p1 = jnp.tile(b1[half:], 4).reshape(1, 128)
    b2p = jnp.tile(b2, 4).reshape(1, 4 * d_out)

    out_p = pl.pallas_call(
        _epi_body,
        grid=(g,),
        in_specs=[
            pl.BlockSpec((pb // 4, 128), lambda j: (j, 0)),
            pl.BlockSpec((pb // 4, 128), lambda j: (j + off1, 0)),
            pl.BlockSpec((pb // 4, 128), lambda j: (j, 0)),
            pl.BlockSpec((pb // 4, 128), lambda j: (j, 0)),
            pl.BlockSpec((pb // 4, 128), lambda j: (j, 0)),
            pl.BlockSpec((1, 128), lambda j: (0, 0)),
            pl.BlockSpec((1, 128), lambda j: (0, 0)),
            pl.BlockSpec((128, 4 * d_out), lambda j: (0, 0)),
            pl.BlockSpec((128, 4 * d_out), lambda j: (0, 0)),
            pl.BlockSpec((1, 4 * d_out), lambda j: (0, 0)),
        ],
        out_specs=pl.BlockSpec((pb // 4, 4 * d_out), lambda j: (j, 0)),
        out_shape=jax.ShapeDtypeStruct((prows, 4 * d_out), jnp.float32),
    )(s_lin, s_lin, hsp0, hsp1, disp, b1p0, b1p1, w2b0, w2b1, b2p)

    return out_p.reshape(np_pad, d_out)[:n]


# speedup vs baseline: 49.5640x; 1.1053x over previous
import functools

import jax
import jax.numpy as jnp
from jax import lax
from jax.experimental import pallas as pl
from jax.experimental.pallas import tpu as pltpu
from jax.experimental.pallas import tpu_sc as plsc

NC = 2
NS = 16
LANES = 128


def _deg_body(dst2, degp, acc, zbuf, ones, didx, sem):
    c = lax.axis_index("c")
    s = lax.axis_index("s")
    wid = s * NC + c
    acc_rows = acc.shape[0]
    zt = zbuf.shape[0]

    z16 = jnp.zeros((16,), jnp.float32)

    def zb(i, _):
        zbuf[pl.ds(i * 16, 16)] = z16
        return 0

    lax.fori_loop(0, zt // 16, zb, 0)

    def ob(i, _):
        ones[pl.ds(i * 16, 16)] = z16 + 1.0
        return 0

    lax.fori_loop(0, 128 // 16, ob, 0)

    pltpu.sync_copy(zbuf, acc.at[pl.ds(s * zt, zt)])
    plsc.subcore_barrier()

    rows_per_tile = dst2.shape[0] // (NC * NS)
    row0 = wid * rows_per_tile
    ng = rows_per_tile // 4

    pltpu.sync_copy(dst2.at[pl.ds(row0, 4)], didx.at[0])

    def outer(g, _):
        p = lax.rem(g, 2)
        q = 1 - p

        @pl.when(g + 1 < ng)
        def _():
            pltpu.async_copy(dst2.at[pl.ds(row0 + (g + 1) * 4, 4)],
                             didx.at[q], sem)

        for j in range(4):
            pltpu.sync_copy(ones, acc.at[didx.at[p, j]], add=True)

        @pl.when(g + 1 < ng)
        def _():
            pltpu.make_async_copy(dst2.at[pl.ds(0, 4)], didx.at[q],
                                  sem).wait()
        return 0

    lax.fori_loop(0, ng, outer, 0)
    plsc.subcore_barrier()

    wt = acc_rows // NS

    @pl.when(c == 0)
    def _():
        pltpu.sync_copy(acc.at[pl.ds(s * wt, wt)], degp.at[0, pl.ds(s * wt, wt)])

    @pl.when(c == 1)
    def _():
        pltpu.sync_copy(acc.at[pl.ds(s * wt, wt)], degp.at[1, pl.ds(s * wt, wt)])


def _gather_body(hs0, hs1, src2, dst2, out_s, acc, rows, sidx, didx,
                 gsem0, gsem1, gsem2, gsem3, isem):
    c = lax.axis_index("c")
    s = lax.axis_index("s")
    acc_rows = acc.shape[0]

    z16 = jnp.zeros((16,), jnp.float32)

    def zrow(r, _):
        rows[0, r, pl.ds(0, 16)] = z16
        rows[0, r, pl.ds(16, 16)] = z16
        return 0

    lax.fori_loop(0, 128, zrow, 0)

    zt = acc_rows // NS

    def zcopy(i, _):
        pltpu.sync_copy(rows.at[0], acc.at[pl.ds(s * zt + i * 128, 128)])
        return 0

    lax.fori_loop(0, zt // 128, zcopy, 0)
    plsc.subcore_barrier()

    rows_per_tile = src2.shape[0] // NS
    row0 = s * rows_per_tile
    ng = rows_per_tile // 8
    sems = (gsem0, gsem1, gsem2, gsem3)

    def run(hs_ref):
        pltpu.sync_copy(src2.at[pl.ds(row0, 8)], sidx.at[0])
        pltpu.sync_copy(dst2.at[pl.ds(row0, 8)], didx.at[0])
        pltpu.async_copy(hs_ref.at[sidx.at[0, 0]], rows.at[0], sems[0])
        pltpu.async_copy(hs_ref.at[sidx.at[0, 1]], rows.at[1], sems[1])
        pltpu.async_copy(hs_ref.at[sidx.at[0, 2]], rows.at[2], sems[2])

        def group(g, _):
            p = lax.rem(g, 2)
            q = 1 - p

            @pl.when(g + 1 < ng)
            def _():
                pltpu.async_copy(src2.at[pl.ds(row0 + (g + 1) * 8, 8)],
                                 sidx.at[q], isem)
                pltpu.async_copy(dst2.at[pl.ds(row0 + (g + 1) * 8, 8)],
                                 didx.at[q], isem)

            for j in range(8):
                jb = j & 3
                sem = sems[jb]
                pltpu.make_async_copy(hs_ref.at[sidx.at[p, j]],
                                      rows.at[jb], sem).wait()
                nsem = sems[(j + 3) & 3]
                if j < 5:
                    pltpu.async_copy(hs_ref.at[sidx.at[p, j + 3]],
                                     rows.at[(j + 3) & 3], nsem)
                elif j == 5:
                    @pl.when(g + 1 < ng)
                    def _():
                        pltpu.make_async_copy(src2.at[pl.ds(0, 8)],
                                              sidx.at[q], isem).wait()
                        pltpu.make_async_copy(dst2.at[pl.ds(0, 8)],
                                              didx.at[q], isem).wait()
                        pltpu.async_copy(hs_ref.at[sidx.at[q, 0]],
                                         rows.at[(j + 3) & 3], nsem)
                else:
                    jn = j - 5
                    @pl.when(g + 1 < ng)
                    def _():
                        pltpu.async_copy(hs_ref.at[sidx.at[q, jn]],
                                         rows.at[(j + 3) & 3], nsem)
                pltpu.sync_copy(rows.at[jb], acc.at[didx.at[p, j]], add=True)
            return 0

        lax.fori_loop(0, ng, group, 0)

    @pl.when(c == 0)
    def _():
        run(hs0)

    @pl.when(c == 1)
    def _():
        run(hs1)

    plsc.subcore_barrier()
    wt = acc_rows // NS

    @pl.when(c == 0)
    def _():
        pltpu.sync_copy(acc.at[pl.ds(s * wt, wt)],
                        out_s.at[0, pl.ds(s * wt, wt)])

    @pl.when(c == 1)
    def _():
        pltpu.sync_copy(acc.at[pl.ds(s * wt, wt)],
                        out_s.at[1, pl.ds(s * wt, wt)])


def _proj_body(xp_ref, w0_ref, w1_ref, dp_ref, hs0_ref, hs1_ref):
    xb = xp_ref[...]
    d = dp_ref[...]
    hs0_ref[...] = d * jnp.dot(xb, w0_ref[...],
                               preferred_element_type=jnp.float32)
    hs1_ref[...] = d * jnp.dot(xb, w1_ref[...],
                               preferred_element_type=jnp.float32)


def _epi_body(s0_ref, s1_ref, h0_ref, h1_ref, dp_ref, b10_ref, b11_ref,
              w20_ref, w21_ref, b2p_ref, out_ref):
    d = dp_ref[...]
    t0 = jnp.maximum((s0_ref[...] + h0_ref[...]) * d + b10_ref[...], 0.0)
    t1 = jnp.maximum((s1_ref[...] + h1_ref[...]) * d + b11_ref[...], 0.0)
    out_ref[...] = (
        jnp.dot(t0, w20_ref[...], preferred_element_type=jnp.float32)
        + jnp.dot(t1, w21_ref[...], preferred_element_type=jnp.float32)
        + b2p_ref[...])


def kernel(x, edge_index, W1, b1, W2, b2):
    n = x.shape[0]
    e = edge_index.shape[1]
    d_hid = W1.shape[1]
    d_out = W2.shape[1]
    half = d_hid // 2

    group = LANES * NS * 8
    ep = ((e + group - 1) // group) * group
    acc_rows = ((n + 1 + NS * 128 - 1) // (NS * 128)) * (NS * 128)

    src = edge_index[0]
    dst = edge_index[1]
    pad = ep - e
    src_p = jnp.concatenate([src, jnp.zeros((pad,), jnp.int32)])
    dst_p = jnp.concatenate([dst, jnp.full((pad,), n, jnp.int32)])
    src2 = src_p.reshape(ep // LANES, LANES)
    dst2 = dst_p.reshape(ep // LANES, LANES)

    mesh = plsc.VectorSubcoreMesh(core_axis_name="c", subcore_axis_name="s")
    sc_params = pltpu.CompilerParams(use_tc_tiling_on_sc=False)

    degp = pl.kernel(
        _deg_body,
        out_type=jax.ShapeDtypeStruct((2, acc_rows), jnp.float32),
        mesh=mesh,
        scratch_types=[
            pltpu.VMEM_SHARED((acc_rows,), jnp.float32),
            pltpu.VMEM((acc_rows // NS,), jnp.float32),
            pltpu.VMEM((LANES,), jnp.float32),
            pltpu.VMEM((2, 4, LANES), jnp.int32),
            pltpu.SemaphoreType.DMA,
        ],
        compiler_params=sc_params,
    )(dst2)

    pb = 1024
    g = (n + pb - 1) // pb
    np_pad = g * pb
    prows = np_pad // 4

    deg = degp[0, :np_pad] + degp[1, :np_pad] + 1.0
    disp = jnp.repeat(lax.rsqrt(deg), half).reshape(prows, 128)

    x_p = x.reshape(n // 4, 4 * x.shape[1])
    eye4 = jnp.eye(4, dtype=jnp.float32)
    w1b0 = jnp.kron(eye4, W1[:, :half])
    w1b1 = jnp.kron(eye4, W1[:, half:])

    hsp0, hsp1 = pl.pallas_call(
        _proj_body,
        grid=(g,),
        in_specs=[
            pl.BlockSpec((pb // 4, 4 * x.shape[1]), lambda j: (j, 0)),
            pl.BlockSpec((4 * x.shape[1], 128), lambda j: (0, 0)),
            pl.BlockSpec((4 * x.shape[1], 128), lambda j: (0, 0)),
            pl.BlockSpec((pb // 4, 128), lambda j: (j, 0)),
        ],
        out_specs=[
            pl.BlockSpec((pb // 4, 128), lambda j: (j, 0)),
            pl.BlockSpec((pb // 4, 128), lambda j: (j, 0)),
        ],
        out_shape=[
            jax.ShapeDtypeStruct((prows, 128), jnp.float32),
            jax.ShapeDtypeStruct((prows, 128), jnp.float32),
        ],
    )(x_p, w1b0, w1b1, disp)

    hs0_lin = hsp0.reshape(np_pad, half)
    hs1_lin = hsp1.reshape(np_pad, half)

    s_agg = pl.kernel(
        _gather_body,
        out_type=jax.ShapeDtypeStruct((2, acc_rows, half), jnp.float32),
        mesh=mesh,
        scratch_types=[
            pltpu.VMEM_SHARED((acc_rows, half), jnp.float32),
            pltpu.VMEM((4, LANES, half), jnp.float32),
            pltpu.VMEM((2, 8, LANES), jnp.int32),
            pltpu.VMEM((2, 8, LANES), jnp.int32),
            pltpu.SemaphoreType.DMA,
            pltpu.SemaphoreType.DMA,
            pltpu.SemaphoreType.DMA,
            pltpu.SemaphoreType.DMA,
            pltpu.SemaphoreType.DMA,
        ],
        compiler_params=sc_params,
    )(hs0_lin, hs1_lin, src2, dst2)

    s_lin = s_agg.reshape(2 * acc_rows * half // 128, 128)
    off1 = acc_rows * half // 128 // (pb // 4)

    w2b0 = jnp.kron(eye4, W2[:half, :])
    w2b1 = jnp.kron(eye4, W2[half:, :])
    b1p0 = jnp.tile(b1[:half], 4).reshape(1, 128)
    b1p1 = jnp.tile(b1[half:], 4).reshape(1, 128)
    b2p = jnp.tile(b2, 4).reshape(1, 4 * d_out)

    out_p = pl.pallas_call(
        _epi_body,
        grid=(g,),
        in_specs=[
            pl.BlockSpec((pb // 4, 128), lambda j: (j, 0)),
            pl.BlockSpec((pb // 4, 128), lambda j: (j + off1, 0)),
            pl.BlockSpec((pb // 4, 128), lambda j: (j, 0)),
            pl.BlockSpec((pb // 4, 128), lambda j: (j, 0)),
            pl.BlockSpec((pb // 4, 128), lambda j: (j, 0)),
            pl.BlockSpec((1, 128), lambda j: (0, 0)),
            pl.BlockSpec((1, 128), lambda j: (0, 0)),
            pl.BlockSpec((128, 4 * d_out), lambda j: (0, 0)),
            pl.BlockSpec((128, 4 * d_out), lambda j: (0, 0)),
            pl.BlockSpec((1, 4 * d_out), lambda j: (0, 0)),
        ],
        out_specs=pl.BlockSpec((pb // 4, 4 * d_out), lambda j: (j, 0)),
        out_shape=jax.ShapeDtypeStruct((prows, 4 * d_out), jnp.float32),
    )(s_lin, s_lin, hsp0, hsp1, disp, b1p0, b1p1, w2b0, w2b1, b2p)

    return out_p.reshape(np_pad, d_out)[:n]
